# baseline (speedup 1.0000x reference)
import jax
import jax.numpy as jnp
from jax import lax
from jax.experimental import pallas as pl
from jax.experimental.pallas import tpu as pltpu
from jax.experimental.pallas import tpu_sc as plsc

N = 10000
E = 320000
D = 128
G = 64

NC = 2
NS = 16
NW = NC * NS
CHUNK = 128
EPAD = 327680
NROWS = EPAD // CHUNK
CPW = NROWS // NW
NPAD = 10112
RPT = NPAD // NS
NB = 3
NGRP = CPW // NB
NREM = CPW - NB * NGRP

_MESH = plsc.VectorSubcoreMesh(core_axis_name="c", subcore_axis_name="s")


def _gather(hw, ib, buf, sem):
    return pltpu.async_copy(hw.at[ib.at[0]], buf, sem)


def _gather_wait(hw, ib, buf, sem):
    pltpu.make_async_copy(hw.at[ib.at[0]], buf, sem).wait()


def _scat(buf, acc, ib, sem):
    pltpu.async_copy(buf, acc.at[ib.at[1]], sem, add=True)


def _scat_wait(buf, acc, ib, sem):
    pltpu.make_async_copy(buf, acc.at[ib.at[1]], sem).wait()


def _agg_body(hw, eidx, zeros, out,
              ib0, ib1, ib2, buf0, buf1, buf2, acc,
              g0, g1, g2, s0, s1, s2):
    cid = lax.axis_index("c")
    sid = lax.axis_index("s")
    wid = cid * NS + sid

    base = wid * CPW

    pltpu.sync_copy(eidx.at[base + 0], ib0)
    _gather(hw, ib0, buf0, g0)
    pltpu.sync_copy(eidx.at[base + 1], ib1)
    _gather(hw, ib1, buf1, g1)
    pltpu.sync_copy(eidx.at[base + 2], ib2)
    _gather(hw, ib2, buf2, g2)

    pltpu.sync_copy(zeros, acc.at[pl.ds(sid * RPT, RPT)])
    plsc.subcore_barrier()

    def step(i, carry):
        j = NB * i
        _gather_wait(hw, ib0, buf0, g0)
        _scat(buf0, acc, ib0, s0)
        _gather_wait(hw, ib1, buf1, g1)
        _scat(buf1, acc, ib1, s1)
        _gather_wait(hw, ib2, buf2, g2)
        _scat(buf2, acc, ib2, s2)

        @pl.when(j + 3 < CPW)
        def _():
            _scat_wait(buf0, acc, ib0, s0)
            pltpu.sync_copy(eidx.at[base + j + 3], ib0)
            _gather(hw, ib0, buf0, g0)

        @pl.when(j + 4 < CPW)
        def _():
            _scat_wait(buf1, acc, ib1, s1)
            pltpu.sync_copy(eidx.at[base + j + 4], ib1)
            _gather(hw, ib1, buf1, g1)

        @pl.when(j + 5 < CPW)
        def _():
            _scat_wait(buf2, acc, ib2, s2)
            pltpu.sync_copy(eidx.at[base + j + 5], ib2)
            _gather(hw, ib2, buf2, g2)

        return carry

    lax.fori_loop(0, NGRP, step, 0)

    _gather_wait(hw, ib0, buf0, g0)
    _scat(buf0, acc, ib0, s0)
    _gather_wait(hw, ib1, buf1, g1)
    _scat(buf1, acc, ib1, s1)
    _scat_wait(buf0, acc, ib0, s0)
    _scat_wait(buf1, acc, ib1, s1)

    plsc.subcore_barrier()
    pltpu.sync_copy(acc.at[pl.ds(sid * RPT, RPT)], out.at[cid, pl.ds(sid * RPT, RPT)])


_agg = pl.kernel(
    _agg_body,
    out_type=jax.ShapeDtypeStruct((NC, NPAD, D), jnp.float32),
    mesh=_MESH,
    scratch_types=[
        pltpu.VMEM((2, CHUNK), jnp.int32),
        pltpu.VMEM((2, CHUNK), jnp.int32),
        pltpu.VMEM((2, CHUNK), jnp.int32),
        pltpu.VMEM((CHUNK, D), jnp.float32),
        pltpu.VMEM((CHUNK, D), jnp.float32),
        pltpu.VMEM((CHUNK, D), jnp.float32),
        pltpu.VMEM_SHARED((NPAD, D), jnp.float32),
        pltpu.SemaphoreType.DMA,
        pltpu.SemaphoreType.DMA,
        pltpu.SemaphoreType.DMA,
        pltpu.SemaphoreType.DMA,
        pltpu.SemaphoreType.DMA,
        pltpu.SemaphoreType.DMA,
    ],
)

_BLK = 2000


def _mm1_body(x_ref, w_ref, o_ref):
    o_ref[...] = jnp.dot(x_ref[...], w_ref[...],
                         preferred_element_type=jnp.float32)


def _mm1(x, w):
    return pl.pallas_call(
        _mm1_body,
        grid=(N // _BLK,),
        in_specs=[pl.BlockSpec((_BLK, D), lambda i: (i, 0)),
                  pl.BlockSpec((D, D), lambda i: (0, 0))],
        out_specs=pl.BlockSpec((_BLK, D), lambda i: (i, 0)),
        out_shape=jax.ShapeDtypeStruct((N, D), jnp.float32),
    )(x, w)


def _mm2_body(p0_ref, p1_ref, w_ref, o_ref):
    h = jnp.maximum(p0_ref[0] + p1_ref[0], 0.0)
    o_ref[...] = jnp.dot(h, w_ref[...],
                         preferred_element_type=jnp.float32)


def _mm2(pp, w):
    return pl.pallas_call(
        _mm2_body,
        grid=(N // _BLK,),
        in_specs=[pl.BlockSpec((1, _BLK, D), lambda i: (0, i, 0)),
                  pl.BlockSpec((1, _BLK, D), lambda i: (1, i, 0)),
                  pl.BlockSpec((D, D), lambda i: (0, 0))],
        out_specs=pl.BlockSpec((_BLK, D), lambda i: (i, 0)),
        out_shape=jax.ShapeDtypeStruct((N, D), jnp.float32),
    )(pp, pp, w)


def _pool_body(p0_ref, p1_ref, b_ref, w_ref, o_ref):
    h = p0_ref[0] + p1_ref[0]
    gids = lax.broadcasted_iota(jnp.int32, (G, N), 0)
    m = (gids == b_ref[...]).astype(jnp.float32)
    sums = jnp.dot(m, h, preferred_element_type=jnp.float32)
    counts = jnp.sum(m, axis=1, keepdims=True)
    pooled = sums / jnp.maximum(counts, 1.0)
    o_ref[...] = jnp.dot(pooled, w_ref[...],
                         preferred_element_type=jnp.float32)


def _pool(pp, batch2d, wlin):
    return pl.pallas_call(
        _pool_body,
        grid=(1,),
        in_specs=[pl.BlockSpec((1, N, D), lambda i: (0, 0, 0)),
                  pl.BlockSpec((1, N, D), lambda i: (1, 0, 0)),
                  pl.BlockSpec((1, N), lambda i: (0, 0)),
                  pl.BlockSpec((D, D), lambda i: (0, 0))],
        out_specs=pl.BlockSpec((G, D), lambda i: (0, 0)),
        out_shape=jax.ShapeDtypeStruct((G, D), jnp.float32),
    )(pp, pp, batch2d, wlin)


def kernel(x, edge_index, batch, W1, W2, W3, Wlin):
    npad_e = EPAD - E
    pad_src = jnp.arange(npad_e, dtype=jnp.int32) % N
    pad_dst = N + (jnp.arange(npad_e, dtype=jnp.int32) % (NPAD - N))
    src = jnp.concatenate([edge_index[0], pad_src]).reshape(NROWS, CHUNK)
    dst = jnp.concatenate([edge_index[1], pad_dst]).reshape(NROWS, CHUNK)
    eidx = jnp.stack([src, dst], axis=1)
    zeros = jnp.zeros((RPT, D), jnp.float32)
    batch2d = batch.reshape(1, N)
    t = _mm1(x, W1)
    pp = _agg(t, eidx, zeros)
    t = _mm2(pp, W2)
    pp = _agg(t, eidx, zeros)
    t = _mm2(pp, W3)
    pp = _agg(t, eidx, zeros)
    return _pool(pp, batch2d, Wlin)


# speedup vs baseline: 1.0559x; 1.0559x over previous
import jax
import jax.numpy as jnp
from jax import lax
from jax.experimental import pallas as pl
from jax.experimental.pallas import tpu as pltpu
from jax.experimental.pallas import tpu_sc as plsc

N = 10000
E = 320000
D = 128
G = 64

NC = 2
NS = 16
NW = NC * NS
CHUNK = 128
EPAD = 327680
NROWS = EPAD // CHUNK
CPW = NROWS // NW
NPAD = 10112
RPT = NPAD // NS
NB = 3
NGRP = CPW // NB
NREM = CPW - NB * NGRP

_MESH = plsc.VectorSubcoreMesh(core_axis_name="c", subcore_axis_name="s")


def _gather(hw, ib, buf, sem):
    return pltpu.async_copy(hw.at[ib.at[0]], buf, sem)


def _gather_wait(hw, ib, buf, sem):
    pltpu.make_async_copy(hw.at[ib.at[0]], buf, sem).wait()


def _scat(buf, acc, ib, sem):
    pltpu.async_copy(buf, acc.at[ib.at[1]], sem, add=True)


def _scat_wait(buf, acc, ib, sem):
    pltpu.make_async_copy(buf, acc.at[ib.at[1]], sem).wait()


def _agg_body(hw, eidx, zeros, out,
              ib0, ib1, ib2, buf0, buf1, buf2, acc,
              g0, g1, g2, s0, s1, s2):
    cid = lax.axis_index("c")
    sid = lax.axis_index("s")
    wid = cid * NS + sid

    base = wid * CPW

    pltpu.sync_copy(eidx.at[base + 0], ib0)
    _gather(hw, ib0, buf0, g0)
    pltpu.sync_copy(eidx.at[base + 1], ib1)
    _gather(hw, ib1, buf1, g1)
    pltpu.sync_copy(eidx.at[base + 2], ib2)
    _gather(hw, ib2, buf2, g2)

    pltpu.sync_copy(zeros, acc.at[pl.ds(sid * RPT, RPT)])
    plsc.subcore_barrier()

    def step(i, carry):
        j = NB * i
        _gather_wait(hw, ib0, buf0, g0)
        _scat(buf0, acc, ib0, s0)
        _gather_wait(hw, ib1, buf1, g1)
        _scat(buf1, acc, ib1, s1)

        @pl.when(j + 3 < CPW)
        def _():
            _scat_wait(buf0, acc, ib0, s0)
            pltpu.sync_copy(eidx.at[base + j + 3], ib0)
            _gather(hw, ib0, buf0, g0)

        _gather_wait(hw, ib2, buf2, g2)
        _scat(buf2, acc, ib2, s2)

        @pl.when(j + 4 < CPW)
        def _():
            _scat_wait(buf1, acc, ib1, s1)
            pltpu.sync_copy(eidx.at[base + j + 4], ib1)
            _gather(hw, ib1, buf1, g1)

        @pl.when(j + 5 < CPW)
        def _():
            _scat_wait(buf2, acc, ib2, s2)
            pltpu.sync_copy(eidx.at[base + j + 5], ib2)
            _gather(hw, ib2, buf2, g2)

        return carry

    lax.fori_loop(0, NGRP, step, 0)

    _gather_wait(hw, ib0, buf0, g0)
    _scat(buf0, acc, ib0, s0)
    _gather_wait(hw, ib1, buf1, g1)
    _scat(buf1, acc, ib1, s1)
    _scat_wait(buf0, acc, ib0, s0)
    _scat_wait(buf1, acc, ib1, s1)

    plsc.subcore_barrier()
    pltpu.sync_copy(acc.at[pl.ds(sid * RPT, RPT)], out.at[cid, pl.ds(sid * RPT, RPT)])


_agg = pl.kernel(
    _agg_body,
    out_type=jax.ShapeDtypeStruct((NC, NPAD, D), jnp.float32),
    mesh=_MESH,
    scratch_types=[
        pltpu.VMEM((2, CHUNK), jnp.int32),
        pltpu.VMEM((2, CHUNK), jnp.int32),
        pltpu.VMEM((2, CHUNK), jnp.int32),
        pltpu.VMEM((CHUNK, D), jnp.float32),
        pltpu.VMEM((CHUNK, D), jnp.float32),
        pltpu.VMEM((CHUNK, D), jnp.float32),
        pltpu.VMEM_SHARED((NPAD, D), jnp.float32),
        pltpu.SemaphoreType.DMA,
        pltpu.SemaphoreType.DMA,
        pltpu.SemaphoreType.DMA,
        pltpu.SemaphoreType.DMA,
        pltpu.SemaphoreType.DMA,
        pltpu.SemaphoreType.DMA,
    ],
)

_BLK = 2000


def _mm1_body(x_ref, w_ref, o_ref):
    o_ref[...] = jnp.dot(x_ref[...], w_ref[...],
                         preferred_element_type=jnp.float32)


def _mm1(x, w):
    return pl.pallas_call(
        _mm1_body,
        grid=(N // _BLK,),
        in_specs=[pl.BlockSpec((_BLK, D), lambda i: (i, 0)),
                  pl.BlockSpec((D, D), lambda i: (0, 0))],
        out_specs=pl.BlockSpec((_BLK, D), lambda i: (i, 0)),
        out_shape=jax.ShapeDtypeStruct((N, D), jnp.float32),
    )(x, w)


def _mm2_body(p0_ref, p1_ref, w_ref, o_ref):
    h = jnp.maximum(p0_ref[0] + p1_ref[0], 0.0)
    o_ref[...] = jnp.dot(h, w_ref[...],
                         preferred_element_type=jnp.float32)


def _mm2(pp, w):
    return pl.pallas_call(
        _mm2_body,
        grid=(N // _BLK,),
        in_specs=[pl.BlockSpec((1, _BLK, D), lambda i: (0, i, 0)),
                  pl.BlockSpec((1, _BLK, D), lambda i: (1, i, 0)),
                  pl.BlockSpec((D, D), lambda i: (0, 0))],
        out_specs=pl.BlockSpec((_BLK, D), lambda i: (i, 0)),
        out_shape=jax.ShapeDtypeStruct((N, D), jnp.float32),
    )(pp, pp, w)


def _pool_body(p0_ref, p1_ref, b_ref, w_ref, o_ref):
    h = p0_ref[0] + p1_ref[0]
    gids = lax.broadcasted_iota(jnp.int32, (G, N), 0)
    m = (gids == b_ref[...]).astype(jnp.float32)
    sums = jnp.dot(m, h, preferred_element_type=jnp.float32)
    counts = jnp.sum(m, axis=1, keepdims=True)
    pooled = sums / jnp.maximum(counts, 1.0)
    o_ref[...] = jnp.dot(pooled, w_ref[...],
                         preferred_element_type=jnp.float32)


def _pool(pp, batch2d, wlin):
    return pl.pallas_call(
        _pool_body,
        grid=(1,),
        in_specs=[pl.BlockSpec((1, N, D), lambda i: (0, 0, 0)),
                  pl.BlockSpec((1, N, D), lambda i: (1, 0, 0)),
                  pl.BlockSpec((1, N), lambda i: (0, 0)),
                  pl.BlockSpec((D, D), lambda i: (0, 0))],
        out_specs=pl.BlockSpec((G, D), lambda i: (0, 0)),
        out_shape=jax.ShapeDtypeStruct((G, D), jnp.float32),
    )(pp, pp, batch2d, wlin)


def kernel(x, edge_index, batch, W1, W2, W3, Wlin):
    npad_e = EPAD - E
    pad_src = jnp.arange(npad_e, dtype=jnp.int32) % N
    pad_dst = N + (jnp.arange(npad_e, dtype=jnp.int32) % (NPAD - N))
    src = jnp.concatenate([edge_index[0], pad_src]).reshape(NROWS, CHUNK)
    dst = jnp.concatenate([edge_index[1], pad_dst]).reshape(NROWS, CHUNK)
    eidx = jnp.stack([src, dst], axis=1)
    zeros = jnp.zeros((RPT, D), jnp.float32)
    batch2d = batch.reshape(1, N)
    t = _mm1(x, W1)
    pp = _agg(t, eidx, zeros)
    t = _mm2(pp, W2)
    pp = _agg(t, eidx, zeros)
    t = _mm2(pp, W3)
    pp = _agg(t, eidx, zeros)
    return _pool(pp, batch2d, Wlin)
